# baseline (speedup 1.0000x reference)
import jax
import jax.numpy as jnp
from jax import lax
from jax.experimental import pallas as pl
from jax.experimental.pallas import tpu as pltpu
from jax.experimental.pallas import tpu_sc as plsc

N = 10000
NP = 10240
E = 320000
NC = 2
NS = 16
NW = NC * NS
CH = 128
IGRP = 16
CHUNKS = 80
NIG = CHUNKS // IGRP
EP = NW * CHUNKS * CH
FCH = EP // (NS * CH)
FNIG = FCH // IGRP
RPT = NP // NS

_f32 = jnp.float32


def _sc_mesh():
    return plsc.VectorSubcoreMesh(
        core_axis_name="c", subcore_axis_name="s", num_cores=NC, num_subcores=NS
    )


def _deg_body(dst_hbm, z_hbm, out_hbm, dst_v, ones_v, acc, sem):
    c = lax.axis_index("c")
    s = lax.axis_index("s")
    wid = c * NS + s
    r0 = s * RPT
    pltpu.sync_copy(z_hbm.at[pl.ds(r0, RPT)], acc.at[pl.ds(r0, RPT)])
    pltpu.sync_copy(dst_hbm.at[wid], dst_v)
    for i in range(CH // 16):
        ones_v[pl.ds(i * 16, 16)] = jnp.ones((16,), _f32)
    plsc.subcore_barrier()

    @pl.loop(0, CHUNKS)
    def _(j):
        pltpu.sync_copy(ones_v, acc.at[dst_v.at[j]], add=True)

    plsc.subcore_barrier()
    pltpu.sync_copy(acc.at[pl.ds(r0, RPT)], out_hbm.at[c, pl.ds(r0, RPT)])


def _deg_partials(dst_blk, zeros1):
    return pl.kernel(
        _deg_body,
        out_type=jax.ShapeDtypeStruct((NC, NP), _f32),
        mesh=_sc_mesh(),
        scratch_types=[
            pltpu.VMEM((CHUNKS, CH), jnp.int32),
            pltpu.VMEM((CH,), _f32),
            pltpu.VMEM_SHARED((NP,), _f32),
            pltpu.SemaphoreType.DMA,
        ],
    )(dst_blk, zeros1)


def _ring(g_hbm, acc, idx_p, rows, gsem, ssem, nb):
    for b in range(nb):
        pltpu.async_copy(g_hbm.at[idx_p.at[0, b]], rows[b], gsem[b])

    @pl.loop(0, IGRP // nb - 1)
    def _(grp):
        base = grp * nb
        for b in range(nb):
            pltpu.make_async_copy(g_hbm.at[idx_p.at[0, 0]], rows[b], gsem[b]).wait()
            pltpu.async_copy(rows[b], acc.at[idx_p.at[1, base + b]], ssem[b], add=True)
        for b in range(nb):
            pltpu.make_async_copy(rows[b], acc.at[idx_p.at[1, 0]], ssem[b]).wait()
            pltpu.async_copy(g_hbm.at[idx_p.at[0, base + nb + b]], rows[b], gsem[b])

    last = IGRP - nb
    for b in range(nb):
        pltpu.make_async_copy(g_hbm.at[idx_p.at[0, 0]], rows[b], gsem[b]).wait()
        pltpu.async_copy(rows[b], acc.at[idx_p.at[1, last + b]], ssem[b], add=True)
    for b in range(nb):
        pltpu.make_async_copy(rows[b], acc.at[idx_p.at[1, 0]], ssem[b]).wait()


def _agg_main(sdf_hbm, g_hbm, z_hbm, idx, rows, acc, isem, gsem, ssem, nb):
    c = lax.axis_index("c")
    s = lax.axis_index("s")
    r0 = s * RPT
    pltpu.sync_copy(z_hbm.at[pl.ds(r0, RPT)], acc.at[pl.ds(r0, RPT)])
    pltpu.sync_copy(sdf_hbm.at[c, s, 0], idx[0])
    plsc.subcore_barrier()

    for ig in range(FNIG):
        p = ig % 2
        if ig + 1 < FNIG:
            pltpu.async_copy(sdf_hbm.at[c, s, ig + 1], idx[1 - p], isem[1 - p])
        _ring(g_hbm, acc, idx[p], rows, gsem, ssem, nb)
        if ig + 1 < FNIG:
            pltpu.make_async_copy(sdf_hbm.at[c, s, 0], idx[1 - p], isem[1 - p]).wait()

    plsc.subcore_barrier()


def _make_agg_body(nb):
    def _body(sdf_hbm, g_hbm, z_hbm, out_hbm, idx, rows, acc, isem, gsem, ssem):
        c = lax.axis_index("c")
        s = lax.axis_index("s")
        r0 = s * RPT
        _agg_main(sdf_hbm, g_hbm, z_hbm, idx, rows, acc, isem, gsem, ssem, nb)
        pltpu.sync_copy(acc.at[pl.ds(r0, RPT)], out_hbm.at[c, pl.ds(r0, RPT)])

    return _body


def _make_agg_spmem_body(nb):
    def _body(sd_hbm, g_hbm, z_hbm, out_hbm, idx, rows, acc, g_sp, isem, gsem, ssem):
        c = lax.axis_index("c")
        s = lax.axis_index("s")
        r0 = s * RPT
        pltpu.sync_copy(z_hbm.at[pl.ds(r0, RPT)], acc.at[pl.ds(r0, RPT)])
        pltpu.sync_copy(g_hbm.at[pl.ds(c * NP + r0, RPT)], g_sp.at[pl.ds(r0, RPT)])
        pltpu.sync_copy(sd_hbm.at[s, 0], idx[0])
        plsc.subcore_barrier()

        for ig in range(FNIG):
            p = ig % 2
            if ig + 1 < FNIG:
                pltpu.async_copy(sd_hbm.at[s, ig + 1], idx[1 - p], isem[1 - p])
            _ring(g_sp, acc, idx[p], rows, gsem, ssem, nb)
            if ig + 1 < FNIG:
                pltpu.make_async_copy(sd_hbm.at[s, 0], idx[1 - p], isem[1 - p]).wait()

        plsc.subcore_barrier()
        pltpu.sync_copy(acc.at[pl.ds(r0, RPT)], out_hbm.at[c, pl.ds(r0, RPT)])

    return _body


def _agg_halves_spmem(sd_blk, g_stacked, zeros_d, d):
    nb = 4
    return pl.kernel(
        _make_agg_spmem_body(nb),
        out_type=jax.ShapeDtypeStruct((NC, NP, d), _f32),
        mesh=_sc_mesh(),
        scratch_types=[
            [pltpu.VMEM((2, IGRP, CH), jnp.int32) for _ in range(2)],
            [pltpu.VMEM((CH, d), _f32) for _ in range(nb)],
            pltpu.VMEM_SHARED((NP, d), _f32),
            pltpu.VMEM_SHARED((NP, d), _f32),
            [pltpu.SemaphoreType.DMA for _ in range(2)],
            [pltpu.SemaphoreType.DMA for _ in range(nb)],
            [pltpu.SemaphoreType.DMA for _ in range(nb)],
        ],
        compiler_params=pltpu.CompilerParams(use_tc_tiling_on_sc=False),
    )(sd_blk, g_stacked, zeros_d)


def _make_agg_final_body(nb, dh):
    def _body(sdf_hbm, g_hbm, z_hbm, dinvb_hbm, t2_hbm, out_hbm,
              idx, rows, acc, abuf, dbuf, tbuf, obuf, isem, gsem, ssem):
        c = lax.axis_index("c")
        s = lax.axis_index("s")
        r0 = s * RPT
        _agg_main(sdf_hbm, g_hbm, z_hbm, idx, rows, acc, isem, gsem, ssem, nb)

        for blk in range(RPT // CH):
            rb = r0 + blk * CH
            pltpu.sync_copy(acc.at[pl.ds(rb, CH)], abuf)
            pltpu.sync_copy(dinvb_hbm.at[pl.ds(rb, CH)], dbuf)
            pltpu.sync_copy(t2_hbm.at[c, pl.ds(rb, CH)], tbuf)

            @pl.loop(0, CH)
            def _(r):
                for k in range(dh // 16):
                    sl = pl.ds(k * 16, 16)
                    obuf[r, sl] = dbuf[r, sl] * abuf[r, sl] + tbuf[r, sl]

            pltpu.sync_copy(obuf, out_hbm.at[c, pl.ds(rb, CH)])

    return _body


def _agg_halves(sdf_blk, g_stacked, zeros_d, d):
    nb = 8
    return pl.kernel(
        _make_agg_body(nb),
        out_type=jax.ShapeDtypeStruct((NC, NP, d), _f32),
        mesh=_sc_mesh(),
        scratch_types=[
            [pltpu.VMEM((2, IGRP, CH), jnp.int32) for _ in range(2)],
            [pltpu.VMEM((CH, d), _f32) for _ in range(nb)],
            pltpu.VMEM_SHARED((NP, d), _f32),
            [pltpu.SemaphoreType.DMA for _ in range(2)],
            [pltpu.SemaphoreType.DMA for _ in range(nb)],
            [pltpu.SemaphoreType.DMA for _ in range(nb)],
        ],
        compiler_params=pltpu.CompilerParams(use_tc_tiling_on_sc=False),
    )(sdf_blk, g_stacked, zeros_d)


def _agg_halves_final(sdf_blk, g_stacked, zeros_d, dinvb, t2h, d):
    nb = 8
    return pl.kernel(
        _make_agg_final_body(nb, d),
        out_type=jax.ShapeDtypeStruct((NC, NP, d), _f32),
        mesh=_sc_mesh(),
        scratch_types=[
            [pltpu.VMEM((2, IGRP, CH), jnp.int32) for _ in range(2)],
            [pltpu.VMEM((CH, d), _f32) for _ in range(nb)],
            pltpu.VMEM_SHARED((NP, d), _f32),
            pltpu.VMEM((CH, d), _f32),
            pltpu.VMEM((CH, d), _f32),
            pltpu.VMEM((CH, d), _f32),
            pltpu.VMEM((CH, d), _f32),
            [pltpu.SemaphoreType.DMA for _ in range(2)],
            [pltpu.SemaphoreType.DMA for _ in range(nb)],
            [pltpu.SemaphoreType.DMA for _ in range(nb)],
        ],
        compiler_params=pltpu.CompilerParams(use_tc_tiling_on_sc=False),
    )(sdf_blk, g_stacked, zeros_d, dinvb, t2h)


_R = 1024


def _tca_body(deg0_ref, deg1_ref, x_ref, w_ref, g_ref, dinv_ref):
    i = pl.program_id(0)
    deg = deg0_ref[...] + deg1_ref[...] + 1.0
    rows = i * _R + lax.broadcasted_iota(jnp.int32, (_R, 1), 0)
    dinv = jnp.where(rows < N, lax.rsqrt(deg), 0.0)
    h = jnp.dot(x_ref[...], w_ref[...], preferred_element_type=_f32)
    g = h * dinv
    g_ref[0] = g[:, : g.shape[1] // 2]
    g_ref[1] = g[:, g.shape[1] // 2 :]
    dinv_ref[...] = dinv


def _tc_g1(deg0, deg1, x, w1):
    ic = x.shape[1]
    hid = w1.shape[1]
    return pl.pallas_call(
        _tca_body,
        grid=(NP // _R,),
        in_specs=[
            pl.BlockSpec((_R, 1), lambda i: (i, 0)),
            pl.BlockSpec((_R, 1), lambda i: (i, 0)),
            pl.BlockSpec((_R, ic), lambda i: (i, 0)),
            pl.BlockSpec((ic, hid), lambda i: (0, 0)),
        ],
        out_specs=[
            pl.BlockSpec((NC, _R, hid // 2), lambda i: (0, i, 0)),
            pl.BlockSpec((_R, 1), lambda i: (i, 0)),
        ],
        out_shape=[
            jax.ShapeDtypeStruct((NC, NP, hid // 2), _f32),
            jax.ShapeDtypeStruct((NP, 1), _f32),
        ],
    )(deg0, deg1, x, w1)


def _tcb_body(s_ref, g1_ref, dinv_ref, b1_ref, b2_ref, w2_ref, g2_ref, dinvb_ref, t2_ref):
    s_full = jnp.concatenate([s_ref[0], s_ref[1]], axis=1)
    g1_full = jnp.concatenate([g1_ref[0], g1_ref[1]], axis=1)
    a = dinv_ref[...] * (s_full + g1_full) + b1_ref[...]
    a = jnp.maximum(a, 0.0)
    h2 = jnp.dot(a, w2_ref[...], preferred_element_type=_f32)
    hd = h2 * dinv_ref[...]
    oh2 = hd.shape[1] // 2
    g2_ref[0] = hd[:, :oh2]
    g2_ref[1] = hd[:, oh2:]
    t2 = hd * dinv_ref[...] + b2_ref[...]
    t2_ref[0] = t2[:, :oh2]
    t2_ref[1] = t2[:, oh2:]
    dinvb_ref[...] = jnp.broadcast_to(dinv_ref[...], (dinv_ref.shape[0], oh2))


def _tc_g2(s1h, g1h, dinv, b1, b2, w2):
    hh = g1h.shape[2]
    hid = 2 * hh
    oc = w2.shape[1]
    return pl.pallas_call(
        _tcb_body,
        grid=(NP // _R,),
        in_specs=[
            pl.BlockSpec((NC, _R, hh), lambda i: (0, i, 0)),
            pl.BlockSpec((NC, _R, hh), lambda i: (0, i, 0)),
            pl.BlockSpec((_R, 1), lambda i: (i, 0)),
            pl.BlockSpec((1, hid), lambda i: (0, 0)),
            pl.BlockSpec((1, oc), lambda i: (0, 0)),
            pl.BlockSpec((hid, oc), lambda i: (0, 0)),
        ],
        out_specs=[
            pl.BlockSpec((NC, _R, oc // 2), lambda i: (0, i, 0)),
            pl.BlockSpec((_R, oc // 2), lambda i: (i, 0)),
            pl.BlockSpec((NC, _R, oc // 2), lambda i: (0, i, 0)),
        ],
        out_shape=[
            jax.ShapeDtypeStruct((NC, NP, oc // 2), _f32),
            jax.ShapeDtypeStruct((NP, oc // 2), _f32),
            jax.ShapeDtypeStruct((NC, NP, oc // 2), _f32),
        ],
    )(s1h, g1h, dinv, b1, b2, w2)


def kernel(x, edge_index, W1, b1, W2, b2):
    hid = W1.shape[1]
    oc = W2.shape[1]
    hh = hid // 2
    oh = oc // 2

    ei = edge_index.astype(jnp.int32)
    pad = EP - E
    fill = N + (jnp.arange(pad, dtype=jnp.int32) % (NP - N))
    srcp = jnp.concatenate([ei[0], fill])
    dstp = jnp.concatenate([ei[1], fill])
    dst_blk = dstp.reshape(NW, CHUNKS, CH)
    srcr = srcp.reshape(NS, FNIG, IGRP, CH)
    dstr = dstp.reshape(NS, FNIG, IGRP, CH)
    sd_blk = jnp.stack([srcr, dstr], axis=2)
    sdf_blk = jnp.stack([sd_blk, jnp.stack([srcr + NP, dstr], axis=2)], axis=0)
    xp = jnp.pad(x, ((0, NP - N), (0, 0)))

    zeros1 = jnp.zeros((NP,), _f32)
    zeros_h = jnp.zeros((NP, hh), _f32)
    zeros_o = jnp.zeros((NP, oh), _f32)

    degp = _deg_partials(dst_blk, zeros1)
    deg0 = degp[0].reshape(NP, 1)
    deg1 = degp[1].reshape(NP, 1)

    g1h, dinv = _tc_g1(deg0, deg1, xp, W1)
    s1h = _agg_halves_spmem(sd_blk, g1h.reshape(NC * NP, hh), zeros_h, hh)
    g2h, dinvb, t2h = _tc_g2(s1h, g1h, dinv, b1.reshape(1, hid), b2.reshape(1, oc), W2)
    outh = _agg_halves_final(
        sdf_blk, g2h.reshape(NC * NP, oh), zeros_o, dinvb, t2h, oh
    )
    return jnp.concatenate([outh[0], outh[1]], axis=1)[:N]


# speedup vs baseline: 1.2537x; 1.2537x over previous
import jax
import jax.numpy as jnp
from jax import lax
from jax.experimental import pallas as pl
from jax.experimental.pallas import tpu as pltpu
from jax.experimental.pallas import tpu_sc as plsc

N = 10000
NP = 10240
E = 320000
NC = 2
NS = 16
NW = NC * NS
CH = 128
IGRP = 32
CHUNKS = 80
EP = NW * CHUNKS * CH
FCH = EP // (NS * CH)
FNIG = FCH // IGRP
RPT = NP // NS

_f32 = jnp.float32


def _sc_mesh():
    return plsc.VectorSubcoreMesh(
        core_axis_name="c", subcore_axis_name="s", num_cores=NC, num_subcores=NS
    )


_DNB = 4


def _deg_body(dst_hbm, z_hbm, out_hbm, dst_v, ones_v, acc, dsem):
    c = lax.axis_index("c")
    s = lax.axis_index("s")
    wid = c * NS + s
    r0 = s * RPT
    pltpu.sync_copy(z_hbm.at[pl.ds(r0, RPT)], acc.at[pl.ds(r0, RPT)])
    pltpu.sync_copy(dst_hbm.at[wid], dst_v)
    for i in range(CH // 16):
        ones_v[pl.ds(i * 16, 16)] = jnp.ones((16,), _f32)
    plsc.subcore_barrier()

    for b in range(_DNB):
        pltpu.async_copy(ones_v, acc.at[dst_v.at[b]], dsem[b], add=True)

    @pl.loop(0, CHUNKS // _DNB - 1)
    def _(grp):
        base = grp * _DNB
        for b in range(_DNB):
            pltpu.make_async_copy(ones_v, acc.at[dst_v.at[0]], dsem[b]).wait()
            pltpu.async_copy(ones_v, acc.at[dst_v.at[base + _DNB + b]], dsem[b], add=True)

    for b in range(_DNB):
        pltpu.make_async_copy(ones_v, acc.at[dst_v.at[0]], dsem[b]).wait()

    plsc.subcore_barrier()
    pltpu.sync_copy(acc.at[pl.ds(r0, RPT)], out_hbm.at[c, pl.ds(r0, RPT)])


def _deg_partials(dst_blk, zeros1):
    return pl.kernel(
        _deg_body,
        out_type=jax.ShapeDtypeStruct((NC, NP), _f32),
        mesh=_sc_mesh(),
        scratch_types=[
            pltpu.VMEM((CHUNKS, CH), jnp.int32),
            pltpu.VMEM((CH,), _f32),
            pltpu.VMEM_SHARED((NP,), _f32),
            [pltpu.SemaphoreType.DMA for _ in range(_DNB)],
        ],
    )(dst_blk, zeros1)


def _ring(g_hbm, acc, idx_p, rows, gsem, ssem, nb):
    for b in range(nb):
        pltpu.async_copy(g_hbm.at[idx_p.at[0, b]], rows[b], gsem[b])

    @pl.loop(0, IGRP // nb - 1)
    def _(grp):
        base = grp * nb
        for b in range(nb):
            pltpu.make_async_copy(g_hbm.at[idx_p.at[0, 0]], rows[b], gsem[b]).wait()
            pltpu.async_copy(rows[b], acc.at[idx_p.at[1, base + b]], ssem[b], add=True)
        for b in range(nb):
            pltpu.make_async_copy(rows[b], acc.at[idx_p.at[1, 0]], ssem[b]).wait()
            pltpu.async_copy(g_hbm.at[idx_p.at[0, base + nb + b]], rows[b], gsem[b])

    last = IGRP - nb
    for b in range(nb):
        pltpu.make_async_copy(g_hbm.at[idx_p.at[0, 0]], rows[b], gsem[b]).wait()
        pltpu.async_copy(rows[b], acc.at[idx_p.at[1, last + b]], ssem[b], add=True)
    for b in range(nb):
        pltpu.make_async_copy(rows[b], acc.at[idx_p.at[1, 0]], ssem[b]).wait()


def _agg_main(sdf_hbm, g_hbm, z_hbm, idx, rows, acc, isem, gsem, ssem, nb):
    c = lax.axis_index("c")
    s = lax.axis_index("s")
    r0 = s * RPT
    pltpu.async_copy(z_hbm.at[pl.ds(r0, RPT)], acc.at[pl.ds(r0, RPT)], isem[0])
    pltpu.async_copy(sdf_hbm.at[c, s, 0], idx[0], isem[1])
    pltpu.make_async_copy(z_hbm.at[pl.ds(r0, RPT)], acc.at[pl.ds(r0, RPT)], isem[0]).wait()
    pltpu.make_async_copy(sdf_hbm.at[c, s, 0], idx[0], isem[1]).wait()
    plsc.subcore_barrier()

    for ig in range(FNIG):
        p = ig % 2
        if ig + 1 < FNIG:
            pltpu.async_copy(sdf_hbm.at[c, s, ig + 1], idx[1 - p], isem[1 - p])
        _ring(g_hbm, acc, idx[p], rows, gsem, ssem, nb)
        if ig + 1 < FNIG:
            pltpu.make_async_copy(sdf_hbm.at[c, s, 0], idx[1 - p], isem[1 - p]).wait()

    plsc.subcore_barrier()


def _make_agg_body(nb):
    def _body(sdf_hbm, g_hbm, z_hbm, out_hbm, idx, rows, acc, isem, gsem, ssem):
        c = lax.axis_index("c")
        s = lax.axis_index("s")
        r0 = s * RPT
        _agg_main(sdf_hbm, g_hbm, z_hbm, idx, rows, acc, isem, gsem, ssem, nb)
        pltpu.sync_copy(acc.at[pl.ds(r0, RPT)], out_hbm.at[c, pl.ds(r0, RPT)])

    return _body


def _make_agg_spmem_body(nb):
    def _body(sd_hbm, g_hbm, z_hbm, out_hbm, idx, rows, acc, g_sp, isem, gsem, ssem):
        c = lax.axis_index("c")
        s = lax.axis_index("s")
        r0 = s * RPT
        pltpu.sync_copy(z_hbm.at[pl.ds(r0, RPT)], acc.at[pl.ds(r0, RPT)])
        pltpu.sync_copy(g_hbm.at[pl.ds(c * NP + r0, RPT)], g_sp.at[pl.ds(r0, RPT)])
        pltpu.sync_copy(sd_hbm.at[s, 0], idx[0])
        plsc.subcore_barrier()

        for ig in range(FNIG):
            p = ig % 2
            if ig + 1 < FNIG:
                pltpu.async_copy(sd_hbm.at[s, ig + 1], idx[1 - p], isem[1 - p])
            _ring(g_sp, acc, idx[p], rows, gsem, ssem, nb)
            if ig + 1 < FNIG:
                pltpu.make_async_copy(sd_hbm.at[s, 0], idx[1 - p], isem[1 - p]).wait()

        plsc.subcore_barrier()
        pltpu.sync_copy(acc.at[pl.ds(r0, RPT)], out_hbm.at[c, pl.ds(r0, RPT)])

    return _body


def _agg_halves_spmem(sd_blk, g_stacked, zeros_d, d):
    nb = 4
    return pl.kernel(
        _make_agg_spmem_body(nb),
        out_type=jax.ShapeDtypeStruct((NC, NP, d), _f32),
        mesh=_sc_mesh(),
        scratch_types=[
            [pltpu.VMEM((2, IGRP, CH), jnp.int32) for _ in range(2)],
            [pltpu.VMEM((CH, d), _f32) for _ in range(nb)],
            pltpu.VMEM_SHARED((NP, d), _f32),
            pltpu.VMEM_SHARED((NP, d), _f32),
            [pltpu.SemaphoreType.DMA for _ in range(2)],
            [pltpu.SemaphoreType.DMA for _ in range(nb)],
            [pltpu.SemaphoreType.DMA for _ in range(nb)],
        ],
        compiler_params=pltpu.CompilerParams(use_tc_tiling_on_sc=False),
    )(sd_blk, g_stacked, zeros_d)


def _make_agg_final_body(nb, dh):
    def _body(sdf_hbm, g_hbm, z_hbm, dinvb_hbm, t2_hbm, out_hbm,
              idx, rows, acc, abuf, dbuf, tbuf, obuf, isem, gsem, ssem):
        c = lax.axis_index("c")
        s = lax.axis_index("s")
        r0 = s * RPT
        _agg_main(sdf_hbm, g_hbm, z_hbm, idx, rows, acc, isem, gsem, ssem, nb)

        for blk in range(RPT // CH):
            rb = r0 + blk * CH
            pltpu.sync_copy(acc.at[pl.ds(rb, CH)], abuf)
            pltpu.sync_copy(dinvb_hbm.at[pl.ds(rb, CH)], dbuf)
            pltpu.sync_copy(t2_hbm.at[c, pl.ds(rb, CH)], tbuf)

            @pl.loop(0, CH)
            def _(r):
                for k in range(dh // 16):
                    sl = pl.ds(k * 16, 16)
                    obuf[r, sl] = dbuf[r, sl] * abuf[r, sl] + tbuf[r, sl]

            pltpu.sync_copy(obuf, out_hbm.at[c, pl.ds(rb, CH)])

    return _body


def _agg_halves(sdf_blk, g_stacked, zeros_d, d):
    nb = 8
    return pl.kernel(
        _make_agg_body(nb),
        out_type=jax.ShapeDtypeStruct((NC, NP, d), _f32),
        mesh=_sc_mesh(),
        scratch_types=[
            [pltpu.VMEM((2, IGRP, CH), jnp.int32) for _ in range(2)],
            [pltpu.VMEM((CH, d), _f32) for _ in range(nb)],
            pltpu.VMEM_SHARED((NP, d), _f32),
            [pltpu.SemaphoreType.DMA for _ in range(2)],
            [pltpu.SemaphoreType.DMA for _ in range(nb)],
            [pltpu.SemaphoreType.DMA for _ in range(nb)],
        ],
        compiler_params=pltpu.CompilerParams(use_tc_tiling_on_sc=False),
    )(sdf_blk, g_stacked, zeros_d)


def _agg_halves_final(sdf_blk, g_stacked, zeros_d, dinvb, t2h, d):
    nb = 8
    return pl.kernel(
        _make_agg_final_body(nb, d),
        out_type=jax.ShapeDtypeStruct((NC, NP, d), _f32),
        mesh=_sc_mesh(),
        scratch_types=[
            [pltpu.VMEM((2, IGRP, CH), jnp.int32) for _ in range(2)],
            [pltpu.VMEM((CH, d), _f32) for _ in range(nb)],
            pltpu.VMEM_SHARED((NP, d), _f32),
            pltpu.VMEM((CH, d), _f32),
            pltpu.VMEM((CH, d), _f32),
            pltpu.VMEM((CH, d), _f32),
            pltpu.VMEM((CH, d), _f32),
            [pltpu.SemaphoreType.DMA for _ in range(2)],
            [pltpu.SemaphoreType.DMA for _ in range(nb)],
            [pltpu.SemaphoreType.DMA for _ in range(nb)],
        ],
        compiler_params=pltpu.CompilerParams(use_tc_tiling_on_sc=False),
    )(sdf_blk, g_stacked, zeros_d, dinvb, t2h)


_R = 1024


def _tca_body(deg0_ref, deg1_ref, x_ref, w_ref, g_ref, dinv_ref):
    i = pl.program_id(0)
    deg = deg0_ref[...] + deg1_ref[...] + 1.0
    rows = i * _R + lax.broadcasted_iota(jnp.int32, (_R, 1), 0)
    dinv = jnp.where(rows < N, lax.rsqrt(deg), 0.0)
    h = jnp.dot(x_ref[...], w_ref[...], preferred_element_type=_f32)
    g = h * dinv
    g_ref[0] = g[:, : g.shape[1] // 2]
    g_ref[1] = g[:, g.shape[1] // 2 :]
    dinv_ref[...] = dinv


def _tc_g1(deg0, deg1, x, w1):
    ic = x.shape[1]
    hid = w1.shape[1]
    return pl.pallas_call(
        _tca_body,
        grid=(NP // _R,),
        in_specs=[
            pl.BlockSpec((_R, 1), lambda i: (i, 0)),
            pl.BlockSpec((_R, 1), lambda i: (i, 0)),
            pl.BlockSpec((_R, ic), lambda i: (i, 0)),
            pl.BlockSpec((ic, hid), lambda i: (0, 0)),
        ],
        out_specs=[
            pl.BlockSpec((NC, _R, hid // 2), lambda i: (0, i, 0)),
            pl.BlockSpec((_R, 1), lambda i: (i, 0)),
        ],
        out_shape=[
            jax.ShapeDtypeStruct((NC, NP, hid // 2), _f32),
            jax.ShapeDtypeStruct((NP, 1), _f32),
        ],
    )(deg0, deg1, x, w1)


def _tcb_body(s_ref, g1_ref, dinv_ref, b1_ref, b2_ref, w2_ref, g2_ref, dinvb_ref, t2_ref):
    s_full = jnp.concatenate([s_ref[0], s_ref[1]], axis=1)
    g1_full = jnp.concatenate([g1_ref[0], g1_ref[1]], axis=1)
    a = dinv_ref[...] * (s_full + g1_full) + b1_ref[...]
    a = jnp.maximum(a, 0.0)
    h2 = jnp.dot(a, w2_ref[...], preferred_element_type=_f32)
    hd = h2 * dinv_ref[...]
    oh2 = hd.shape[1] // 2
    g2_ref[0] = hd[:, :oh2]
    g2_ref[1] = hd[:, oh2:]
    t2 = hd * dinv_ref[...] + b2_ref[...]
    t2_ref[0] = t2[:, :oh2]
    t2_ref[1] = t2[:, oh2:]
    dinvb_ref[...] = jnp.broadcast_to(dinv_ref[...], (dinv_ref.shape[0], oh2))


def _tc_g2(s1h, g1h, dinv, b1, b2, w2):
    hh = g1h.shape[2]
    hid = 2 * hh
    oc = w2.shape[1]
    return pl.pallas_call(
        _tcb_body,
        grid=(NP // _R,),
        in_specs=[
            pl.BlockSpec((NC, _R, hh), lambda i: (0, i, 0)),
            pl.BlockSpec((NC, _R, hh), lambda i: (0, i, 0)),
            pl.BlockSpec((_R, 1), lambda i: (i, 0)),
            pl.BlockSpec((1, hid), lambda i: (0, 0)),
            pl.BlockSpec((1, oc), lambda i: (0, 0)),
            pl.BlockSpec((hid, oc), lambda i: (0, 0)),
        ],
        out_specs=[
            pl.BlockSpec((NC, _R, oc // 2), lambda i: (0, i, 0)),
            pl.BlockSpec((_R, oc // 2), lambda i: (i, 0)),
            pl.BlockSpec((NC, _R, oc // 2), lambda i: (0, i, 0)),
        ],
        out_shape=[
            jax.ShapeDtypeStruct((NC, NP, oc // 2), _f32),
            jax.ShapeDtypeStruct((NP, oc // 2), _f32),
            jax.ShapeDtypeStruct((NC, NP, oc // 2), _f32),
        ],
    )(s1h, g1h, dinv, b1, b2, w2)


def kernel(x, edge_index, W1, b1, W2, b2):
    hid = W1.shape[1]
    oc = W2.shape[1]
    hh = hid // 2
    oh = oc // 2

    ei = edge_index.astype(jnp.int32)
    pad = EP - E
    fill = N + (jnp.arange(pad, dtype=jnp.int32) % (NP - N))
    srcp = jnp.concatenate([ei[0], fill])
    dstp = jnp.concatenate([ei[1], fill])
    dst_blk = dstp.reshape(NW, CHUNKS, CH)
    srcr = srcp.reshape(NS, FNIG, IGRP, CH)
    dstr = dstp.reshape(NS, FNIG, IGRP, CH)
    sd_blk = jnp.stack([srcr, dstr], axis=2)
    sdf_blk = jnp.stack([sd_blk, jnp.stack([srcr + NP, dstr], axis=2)], axis=0)
    xp = jnp.pad(x, ((0, NP - N), (0, 0)))

    zeros1 = jnp.zeros((NP,), _f32)
    zeros_h = jnp.zeros((NP, hh), _f32)
    zeros_o = jnp.zeros((NP, oh), _f32)

    degp = _deg_partials(dst_blk, zeros1)
    deg0 = degp[0].reshape(NP, 1)
    deg1 = degp[1].reshape(NP, 1)

    g1h, dinv = _tc_g1(deg0, deg1, xp, W1)
    s1h = _agg_halves(sdf_blk, g1h.reshape(NC * NP, hh), zeros_h, hh)
    g2h, dinvb, t2h = _tc_g2(s1h, g1h, dinv, b1.reshape(1, hid), b2.reshape(1, oc), W2)
    outh = _agg_halves_final(
        sdf_blk, g2h.reshape(NC * NP, oh), zeros_o, dinvb, t2h, oh
    )
    return jnp.concatenate([outh[0], outh[1]], axis=1)[:N]
